# baseline (speedup 1.0000x reference)
import functools

import jax
import jax.numpy as jnp
from jax import lax
from jax.experimental import pallas as pl
from jax.experimental.pallas import tpu as pltpu
from jax.experimental.pallas import tpu_sc as plsc

_N = 8192
_M = 8192
_NT = 1024
_SPLIT = 2048


def _argmin_body(data4t_ref, rec4t_ref, idx_ref):
    s = lax.dot_general(
        data4t_ref[...], rec4t_ref[...],
        dimension_numbers=(((0,), (0,)), ((), ())),
        preferred_element_type=jnp.float32,
    )
    idx_ref[...] = jnp.argmin(s, axis=0).astype(jnp.int32)


def _loss_body(sqa_ref, sqb_ref, out_ref):
    t = jnp.sum(jnp.sqrt(sqa_ref[...])) + jnp.sum(jnp.sqrt(sqb_ref[...]))
    out_ref[...] = (t / _N).reshape(1, 1)


def _argmin_part(data4t, rec4t, row_base, nrows):
    blk_base = row_base // _NT
    return pl.pallas_call(
        _argmin_body,
        grid=(nrows // _NT,),
        in_specs=[
            pl.BlockSpec((4, _M), lambda i: (0, 0)),
            pl.BlockSpec((4, _NT), lambda i: (0, i + blk_base)),
        ],
        out_specs=pl.BlockSpec((_NT,), lambda i: (i,)),
        out_shape=jax.ShapeDtypeStruct((nrows,), jnp.int32),
    )(data4t, rec4t)


def _sc_residuals(dn_cat, rn_cat, idx_part, row_base, nrows):
    info = plsc.get_sparse_core_info()
    nw = info.num_cores * info.num_subcores
    b_per_w = nrows // nw
    n_chunks = b_per_w // 16
    mesh = plsc.VectorSubcoreMesh(core_axis_name="c", subcore_axis_name="s")

    @functools.partial(
        pl.kernel,
        out_type=jax.ShapeDtypeStruct((nrows,), jnp.float32),
        mesh=mesh,
        compiler_params=pltpu.CompilerParams(needs_layout_passes=False),
        scratch_types=[
            pltpu.VMEM((3 * _M,), jnp.float32),
            pltpu.VMEM((b_per_w,), jnp.float32),
            pltpu.VMEM((b_per_w,), jnp.float32),
            pltpu.VMEM((b_per_w,), jnp.float32),
            pltpu.VMEM((b_per_w,), jnp.int32),
            pltpu.VMEM((b_per_w,), jnp.float32),
            pltpu.SemaphoreType.DMA,
        ],
    )
    def residual_kernel(dnc_h, rnc_h, idx_h, out_h, dnc_v,
                        rnx_v, rny_v, rnz_v, idx_v, out_v, sem):
        wid = lax.axis_index("s") * info.num_cores + lax.axis_index("c")
        base_l = wid * b_per_w
        base_g = row_base + base_l
        copies = [
            pltpu.async_copy(dnc_h, dnc_v, sem),
            pltpu.async_copy(rnc_h.at[pl.ds(base_g, b_per_w)], rnx_v, sem),
            pltpu.async_copy(rnc_h.at[pl.ds(_N + base_g, b_per_w)],
                             rny_v, sem),
            pltpu.async_copy(rnc_h.at[pl.ds(2 * _N + base_g, b_per_w)],
                             rnz_v, sem),
            pltpu.async_copy(idx_h.at[pl.ds(base_l, b_per_w)], idx_v, sem),
        ]
        for c in copies:
            c.wait()
        for c in range(n_chunks):
            sl = pl.ds(c * 16, 16)
            iv = idx_v[sl]
            dx = plsc.load_gather(dnc_v, [iv]) - rnx_v[sl]
            dy = plsc.load_gather(dnc_v, [iv + _M]) - rny_v[sl]
            dz = plsc.load_gather(dnc_v, [iv + 2 * _M]) - rnz_v[sl]
            out_v[sl] = dx * dx + dy * dy + dz * dz
        pltpu.sync_copy(out_v, out_h.at[pl.ds(base_l, b_per_w)])

    return residual_kernel(dn_cat, rn_cat, idx_part)


def kernel(rec, data, rec_normals, data_normals):
    rec2 = rec[0]
    data2 = data[0]
    rn = rec_normals[0]
    dn = data_normals[0]

    y2 = jnp.sum(data2 * data2, axis=1)
    data4t = jnp.concatenate(
        [-2.0 * data2.T, y2[None, :]], axis=0)
    rec4t = jnp.concatenate(
        [rec2.T, jnp.ones((1, _N), jnp.float32)], axis=0)

    dn_cat = jnp.concatenate([dn[:, 0], dn[:, 1], dn[:, 2]])
    rn_cat = jnp.concatenate([rn[:, 0], rn[:, 1], rn[:, 2]])

    nb = _N - _SPLIT
    idx_a = _argmin_part(data4t, rec4t, 0, _SPLIT)
    sq_a = _sc_residuals(dn_cat, rn_cat, idx_a, 0, _SPLIT)
    idx_b = _argmin_part(data4t, rec4t, _SPLIT, nb)
    sq_b = _sc_residuals(dn_cat, rn_cat, idx_b, _SPLIT, nb)

    loss = pl.pallas_call(
        _loss_body,
        in_specs=[
            pl.BlockSpec((_SPLIT // 128, 128), lambda: (0, 0)),
            pl.BlockSpec((nb // 128, 128), lambda: (0, 0)),
        ],
        out_specs=pl.BlockSpec((1, 1), lambda: (0, 0)),
        out_shape=jax.ShapeDtypeStruct((1, 1), jnp.float32),
    )(sq_a.reshape(_SPLIT // 128, 128), sq_b.reshape(nb // 128, 128))
    return loss[0, 0]


# speedup vs baseline: 1.0170x; 1.0170x over previous
import functools

import jax
import jax.numpy as jnp
from jax import lax
from jax.experimental import pallas as pl
from jax.experimental.pallas import tpu as pltpu
from jax.experimental.pallas import tpu_sc as plsc

_N = 8192
_M = 8192
_NT = 1024
_SPLIT = 2048


def _argmin_body(data4t_ref, rec4t_ref, idx_ref):
    s = lax.dot_general(
        data4t_ref[...], rec4t_ref[...],
        dimension_numbers=(((0,), (0,)), ((), ())),
        preferred_element_type=jnp.float32,
    )
    idx_ref[...] = jnp.argmin(s, axis=0).astype(jnp.int32)


def _loss_body(sqa_ref, sqb_ref, out_ref):
    t = jnp.sum(jnp.sqrt(sqa_ref[...])) + jnp.sum(jnp.sqrt(sqb_ref[...]))
    out_ref[...] = (t / _N).reshape(1, 1)


def _argmin_part(data4t, rec4t, row_base, nrows):
    blk_base = row_base // _NT
    return pl.pallas_call(
        _argmin_body,
        grid=(nrows // _NT,),
        in_specs=[
            pl.BlockSpec((4, _M), lambda i: (0, 0)),
            pl.BlockSpec((4, _NT), lambda i: (0, i + blk_base)),
        ],
        out_specs=pl.BlockSpec((_NT,), lambda i: (i,)),
        out_shape=jax.ShapeDtypeStruct((nrows,), jnp.int32),
    )(data4t, rec4t)


def _sc_residuals(dn_planes, rn_planes, idx_part, row_base, nrows):
    info = plsc.get_sparse_core_info()
    nw = info.num_cores * info.num_subcores
    b_per_w = nrows // nw
    n_chunks = b_per_w // 16
    mesh = plsc.VectorSubcoreMesh(core_axis_name="c", subcore_axis_name="s")

    @functools.partial(
        pl.kernel,
        out_type=jax.ShapeDtypeStruct((nrows,), jnp.float32),
        mesh=mesh,
        compiler_params=pltpu.CompilerParams(needs_layout_passes=False),
        scratch_types=[
            pltpu.VMEM((_M,), jnp.float32),
            pltpu.VMEM((_M,), jnp.float32),
            pltpu.VMEM((_M,), jnp.float32),
            pltpu.VMEM((b_per_w,), jnp.float32),
            pltpu.VMEM((b_per_w,), jnp.float32),
            pltpu.VMEM((b_per_w,), jnp.float32),
            pltpu.VMEM((b_per_w,), jnp.int32),
            pltpu.VMEM((b_per_w,), jnp.float32),
            pltpu.SemaphoreType.DMA,
        ],
    )
    def residual_kernel(dnx_h, dny_h, dnz_h, rnx_h, rny_h, rnz_h, idx_h,
                        out_h, dnx_v, dny_v, dnz_v, rnx_v, rny_v, rnz_v,
                        idx_v, out_v, sem):
        wid = lax.axis_index("s") * info.num_cores + lax.axis_index("c")
        base_l = wid * b_per_w
        base_g = row_base + base_l
        copies = [
            pltpu.async_copy(dnx_h, dnx_v, sem),
            pltpu.async_copy(dny_h, dny_v, sem),
            pltpu.async_copy(dnz_h, dnz_v, sem),
            pltpu.async_copy(rnx_h.at[pl.ds(base_g, b_per_w)], rnx_v, sem),
            pltpu.async_copy(rny_h.at[pl.ds(base_g, b_per_w)], rny_v, sem),
            pltpu.async_copy(rnz_h.at[pl.ds(base_g, b_per_w)], rnz_v, sem),
            pltpu.async_copy(idx_h.at[pl.ds(base_l, b_per_w)], idx_v, sem),
        ]
        for c in copies:
            c.wait()
        for c in range(n_chunks):
            sl = pl.ds(c * 16, 16)
            iv = idx_v[sl]
            dx = plsc.load_gather(dnx_v, [iv]) - rnx_v[sl]
            dy = plsc.load_gather(dny_v, [iv]) - rny_v[sl]
            dz = plsc.load_gather(dnz_v, [iv]) - rnz_v[sl]
            out_v[sl] = dx * dx + dy * dy + dz * dz
        pltpu.sync_copy(out_v, out_h.at[pl.ds(base_l, b_per_w)])

    return residual_kernel(*dn_planes, *rn_planes, idx_part)


def kernel(rec, data, rec_normals, data_normals):
    rec2 = rec[0]
    data2 = data[0]
    rn = rec_normals[0]
    dn = data_normals[0]

    y2 = jnp.sum(data2 * data2, axis=1)
    data4t = jnp.concatenate(
        [-2.0 * data2.T, y2[None, :]], axis=0)
    rec4t = jnp.concatenate(
        [rec2.T, jnp.ones((1, _N), jnp.float32)], axis=0)

    dn_planes = (dn[:, 0], dn[:, 1], dn[:, 2])
    rn_planes = (rn[:, 0], rn[:, 1], rn[:, 2])

    nb = _N - _SPLIT
    idx_a = _argmin_part(data4t, rec4t, 0, _SPLIT)
    sq_a = _sc_residuals(dn_planes, rn_planes, idx_a, 0, _SPLIT)
    idx_b = _argmin_part(data4t, rec4t, _SPLIT, nb)
    sq_b = _sc_residuals(dn_planes, rn_planes, idx_b, _SPLIT, nb)

    loss = pl.pallas_call(
        _loss_body,
        in_specs=[
            pl.BlockSpec((_SPLIT // 128, 128), lambda: (0, 0)),
            pl.BlockSpec((nb // 128, 128), lambda: (0, 0)),
        ],
        out_specs=pl.BlockSpec((1, 1), lambda: (0, 0)),
        out_shape=jax.ShapeDtypeStruct((1, 1), jnp.float32),
    )(sq_a.reshape(_SPLIT // 128, 128), sq_b.reshape(nb // 128, 128))
    return loss[0, 0]


# speedup vs baseline: 1.0439x; 1.0264x over previous
import functools

import jax
import jax.numpy as jnp
from jax import lax
from jax.experimental import pallas as pl
from jax.experimental.pallas import tpu as pltpu
from jax.experimental.pallas import tpu_sc as plsc

_N = 8192
_M = 8192
_NT = 1024
_SPLIT = 2048


def _argmin_body(data4t_ref, rec4t_ref, idx_ref):
    s = lax.dot_general(
        data4t_ref[...], rec4t_ref[...],
        dimension_numbers=(((0,), (0,)), ((), ())),
        preferred_element_type=jnp.float32,
    )
    idx_ref[...] = jnp.argmin(s, axis=0).astype(jnp.int32)


def _partial_body(sq_ref, out_ref):
    out_ref[...] = jnp.sum(jnp.sqrt(sq_ref[...])).reshape(1, 1)


def _loss_body(pb_ref, sqa_ref, out_ref):
    t = pb_ref[0, 0] + jnp.sum(jnp.sqrt(sqa_ref[...]))
    out_ref[...] = (t / _N).reshape(1, 1)


def _argmin_part(data4t, rec4t, row_base, nrows):
    blk_base = row_base // _NT
    return pl.pallas_call(
        _argmin_body,
        grid=(nrows // _NT,),
        in_specs=[
            pl.BlockSpec((4, _M), lambda i: (0, 0)),
            pl.BlockSpec((4, _NT), lambda i: (0, i + blk_base)),
        ],
        out_specs=pl.BlockSpec((_NT,), lambda i: (i,)),
        out_shape=jax.ShapeDtypeStruct((nrows,), jnp.int32),
    )(data4t, rec4t)


def _sc_residuals(dn_planes, rn_planes, idx_part, row_base, nrows,
                  n_workers=None):
    info = plsc.get_sparse_core_info()
    nw = info.num_cores * info.num_subcores
    if n_workers is None:
        n_workers = nw
    b_per_w = nrows // n_workers
    n_chunks = b_per_w // 16
    mesh = plsc.VectorSubcoreMesh(core_axis_name="c", subcore_axis_name="s")

    @functools.partial(
        pl.kernel,
        out_type=jax.ShapeDtypeStruct((nrows,), jnp.float32),
        mesh=mesh,
        compiler_params=pltpu.CompilerParams(needs_layout_passes=False),
        scratch_types=[
            pltpu.VMEM((_M,), jnp.float32),
            pltpu.VMEM((_M,), jnp.float32),
            pltpu.VMEM((_M,), jnp.float32),
            pltpu.VMEM((b_per_w,), jnp.float32),
            pltpu.VMEM((b_per_w,), jnp.float32),
            pltpu.VMEM((b_per_w,), jnp.float32),
            pltpu.VMEM((b_per_w,), jnp.int32),
            pltpu.VMEM((b_per_w,), jnp.float32),
            pltpu.SemaphoreType.DMA,
        ],
    )
    def residual_kernel(dnx_h, dny_h, dnz_h, rnx_h, rny_h, rnz_h, idx_h,
                        out_h, dnx_v, dny_v, dnz_v, rnx_v, rny_v, rnz_v,
                        idx_v, out_v, sem):
        wid = lax.axis_index("s") * info.num_cores + lax.axis_index("c")
        base_l = wid * b_per_w
        base_g = row_base + base_l

        @pl.when(wid < n_workers)
        def _work():
            copies = [
            pltpu.async_copy(dnx_h, dnx_v, sem),
            pltpu.async_copy(dny_h, dny_v, sem),
            pltpu.async_copy(dnz_h, dnz_v, sem),
            pltpu.async_copy(rnx_h.at[pl.ds(base_g, b_per_w)], rnx_v, sem),
            pltpu.async_copy(rny_h.at[pl.ds(base_g, b_per_w)], rny_v, sem),
            pltpu.async_copy(rnz_h.at[pl.ds(base_g, b_per_w)], rnz_v, sem),
                pltpu.async_copy(idx_h.at[pl.ds(base_l, b_per_w)], idx_v,
                                 sem),
            ]
            for c in copies:
                c.wait()
            for c in range(n_chunks):
                sl = pl.ds(c * 16, 16)
                iv = idx_v[sl]
                dx = plsc.load_gather(dnx_v, [iv]) - rnx_v[sl]
                dy = plsc.load_gather(dny_v, [iv]) - rny_v[sl]
                dz = plsc.load_gather(dnz_v, [iv]) - rnz_v[sl]
                out_v[sl] = dx * dx + dy * dy + dz * dz
            pltpu.sync_copy(out_v, out_h.at[pl.ds(base_l, b_per_w)])

    return residual_kernel(*dn_planes, *rn_planes, idx_part)


def kernel(rec, data, rec_normals, data_normals):
    rec2 = rec[0]
    data2 = data[0]
    rn = rec_normals[0]
    dn = data_normals[0]

    y2 = jnp.sum(data2 * data2, axis=1)
    data4t = jnp.concatenate(
        [-2.0 * data2.T, y2[None, :]], axis=0)
    rec4t = jnp.concatenate(
        [rec2.T, jnp.ones((1, _N), jnp.float32)], axis=0)

    dn_planes = (dn[:, 0], dn[:, 1], dn[:, 2])
    rn_planes = (rn[:, 0], rn[:, 1], rn[:, 2])

    nb = _N - _SPLIT
    idx_a = _argmin_part(data4t, rec4t, 0, _SPLIT)
    sq_a = _sc_residuals(dn_planes, rn_planes, idx_a, 0, _SPLIT,
                         n_workers=8)
    idx_b = _argmin_part(data4t, rec4t, _SPLIT, nb)
    sq_b = _sc_residuals(dn_planes, rn_planes, idx_b, _SPLIT, nb)

    pb = pl.pallas_call(
        _partial_body,
        in_specs=[pl.BlockSpec((nb // 128, 128), lambda: (0, 0))],
        out_specs=pl.BlockSpec((1, 1), lambda: (0, 0)),
        out_shape=jax.ShapeDtypeStruct((1, 1), jnp.float32),
    )(sq_b.reshape(nb // 128, 128))

    loss = pl.pallas_call(
        _loss_body,
        in_specs=[
            pl.BlockSpec((1, 1), lambda: (0, 0)),
            pl.BlockSpec((_SPLIT // 128, 128), lambda: (0, 0)),
        ],
        out_specs=pl.BlockSpec((1, 1), lambda: (0, 0)),
        out_shape=jax.ShapeDtypeStruct((1, 1), jnp.float32),
    )(pb, sq_a.reshape(_SPLIT // 128, 128))
    return loss[0, 0]


# speedup vs baseline: 1.0581x; 1.0137x over previous
import functools

import jax
import jax.numpy as jnp
from jax import lax
from jax.experimental import pallas as pl
from jax.experimental.pallas import tpu as pltpu
from jax.experimental.pallas import tpu_sc as plsc

_N = 8192
_M = 8192
_NT = 2048
_SPLIT = 2048


def _argmin_body(data4t_ref, rec4t_ref, idx_ref):
    s = lax.dot_general(
        data4t_ref[...], rec4t_ref[...],
        dimension_numbers=(((0,), (0,)), ((), ())),
        preferred_element_type=jnp.float32,
    )
    idx_ref[...] = jnp.argmin(s, axis=0).astype(jnp.int32)


def _partial_body(sq_ref, out_ref):
    out_ref[...] = jnp.sum(jnp.sqrt(sq_ref[...])).reshape(1, 1)


def _loss_body(pb_ref, sqa_ref, out_ref):
    t = pb_ref[0, 0] + jnp.sum(jnp.sqrt(sqa_ref[...]))
    out_ref[...] = (t / _N).reshape(1, 1)


def _argmin_part(data4t, rec4t, row_base, nrows):
    blk_base = row_base // _NT
    return pl.pallas_call(
        _argmin_body,
        grid=(nrows // _NT,),
        in_specs=[
            pl.BlockSpec((4, _M), lambda i: (0, 0)),
            pl.BlockSpec((4, _NT), lambda i: (0, i + blk_base)),
        ],
        out_specs=pl.BlockSpec((_NT,), lambda i: (i,)),
        out_shape=jax.ShapeDtypeStruct((nrows,), jnp.int32),
        compiler_params=pltpu.CompilerParams(
            vmem_limit_bytes=100 * 1024 * 1024),
    )(data4t, rec4t)


def _sc_residuals(dn_planes, rn_planes, idx_part, row_base, nrows,
                  n_workers=None):
    info = plsc.get_sparse_core_info()
    nw = info.num_cores * info.num_subcores
    if n_workers is None:
        n_workers = nw
    b_per_w = nrows // n_workers
    n_chunks = b_per_w // 16
    mesh = plsc.VectorSubcoreMesh(core_axis_name="c", subcore_axis_name="s")

    @functools.partial(
        pl.kernel,
        out_type=jax.ShapeDtypeStruct((nrows,), jnp.float32),
        mesh=mesh,
        compiler_params=pltpu.CompilerParams(needs_layout_passes=False),
        scratch_types=[
            pltpu.VMEM((_M,), jnp.float32),
            pltpu.VMEM((_M,), jnp.float32),
            pltpu.VMEM((_M,), jnp.float32),
            pltpu.VMEM((b_per_w,), jnp.float32),
            pltpu.VMEM((b_per_w,), jnp.float32),
            pltpu.VMEM((b_per_w,), jnp.float32),
            pltpu.VMEM((b_per_w,), jnp.int32),
            pltpu.VMEM((b_per_w,), jnp.float32),
            pltpu.SemaphoreType.DMA,
        ],
    )
    def residual_kernel(dnx_h, dny_h, dnz_h, rnx_h, rny_h, rnz_h, idx_h,
                        out_h, dnx_v, dny_v, dnz_v, rnx_v, rny_v, rnz_v,
                        idx_v, out_v, sem):
        wid = lax.axis_index("s") * info.num_cores + lax.axis_index("c")
        base_l = wid * b_per_w
        base_g = row_base + base_l

        @pl.when(wid < n_workers)
        def _work():
            copies = [
            pltpu.async_copy(dnx_h, dnx_v, sem),
            pltpu.async_copy(dny_h, dny_v, sem),
            pltpu.async_copy(dnz_h, dnz_v, sem),
            pltpu.async_copy(rnx_h.at[pl.ds(base_g, b_per_w)], rnx_v, sem),
            pltpu.async_copy(rny_h.at[pl.ds(base_g, b_per_w)], rny_v, sem),
            pltpu.async_copy(rnz_h.at[pl.ds(base_g, b_per_w)], rnz_v, sem),
                pltpu.async_copy(idx_h.at[pl.ds(base_l, b_per_w)], idx_v,
                                 sem),
            ]
            for c in copies:
                c.wait()
            for c in range(n_chunks):
                sl = pl.ds(c * 16, 16)
                iv = idx_v[sl]
                dx = plsc.load_gather(dnx_v, [iv]) - rnx_v[sl]
                dy = plsc.load_gather(dny_v, [iv]) - rny_v[sl]
                dz = plsc.load_gather(dnz_v, [iv]) - rnz_v[sl]
                out_v[sl] = dx * dx + dy * dy + dz * dz
            pltpu.sync_copy(out_v, out_h.at[pl.ds(base_l, b_per_w)])

    return residual_kernel(*dn_planes, *rn_planes, idx_part)


def kernel(rec, data, rec_normals, data_normals):
    rec2 = rec[0]
    data2 = data[0]
    rn = rec_normals[0]
    dn = data_normals[0]

    y2 = jnp.sum(data2 * data2, axis=1)
    data4t = jnp.concatenate(
        [-2.0 * data2.T, y2[None, :]], axis=0)
    rec4t = jnp.concatenate(
        [rec2.T, jnp.ones((1, _N), jnp.float32)], axis=0)

    dn_planes = (dn[:, 0], dn[:, 1], dn[:, 2])
    rn_planes = (rn[:, 0], rn[:, 1], rn[:, 2])

    nb = _N - _SPLIT
    idx_a = _argmin_part(data4t, rec4t, 0, _SPLIT)
    sq_a = _sc_residuals(dn_planes, rn_planes, idx_a, 0, _SPLIT,
                         n_workers=8)
    idx_b = _argmin_part(data4t, rec4t, _SPLIT, nb)
    sq_b = _sc_residuals(dn_planes, rn_planes, idx_b, _SPLIT, nb)

    pb = pl.pallas_call(
        _partial_body,
        in_specs=[pl.BlockSpec((nb // 128, 128), lambda: (0, 0))],
        out_specs=pl.BlockSpec((1, 1), lambda: (0, 0)),
        out_shape=jax.ShapeDtypeStruct((1, 1), jnp.float32),
    )(sq_b.reshape(nb // 128, 128))

    loss = pl.pallas_call(
        _loss_body,
        in_specs=[
            pl.BlockSpec((1, 1), lambda: (0, 0)),
            pl.BlockSpec((_SPLIT // 128, 128), lambda: (0, 0)),
        ],
        out_specs=pl.BlockSpec((1, 1), lambda: (0, 0)),
        out_shape=jax.ShapeDtypeStruct((1, 1), jnp.float32),
    )(pb, sq_a.reshape(_SPLIT // 128, 128))
    return loss[0, 0]
